# baseline (speedup 1.0000x reference)
import dataclasses
import functools

import jax
import jax.numpy as jnp
from jax import lax
from jax.experimental import pallas as pl
from jax.experimental.pallas import tpu as pltpu
from jax.experimental.pallas import tpu_sc as plsc

NC = 2
NS = 16
NW = NC * NS
LANES = 16
IDX_W = 128


def kernel(user_ids, item_ids, user_factors, item_factors, user_biases,
           item_biases, global_bias):
    B = user_ids.shape[0]
    D = user_factors.shape[1]
    bpw = B // NW
    nchunks = bpw // IDX_W

    uids = user_ids.astype(jnp.int32).reshape(B // IDX_W, IDX_W)
    iids = item_ids.astype(jnp.int32).reshape(B // IDX_W, IDX_W)
    ub_flat = user_biases.reshape(-1)
    ib_flat = item_biases.reshape(-1)
    gb_vec = jnp.broadcast_to(global_bias.reshape(()), (LANES,))

    mesh = plsc.VectorSubcoreMesh(core_axis_name="c", subcore_axis_name="s")

    cp = pltpu.CompilerParams()
    for field, val in (("needs_layout_passes", False),
                       ("use_tc_tiling_on_sc", False)):
        if field in pltpu.CompilerParams.__dataclass_fields__:
            cp = dataclasses.replace(cp, **{field: val})

    @functools.partial(
        pl.kernel,
        out_type=jax.ShapeDtypeStruct((B,), jnp.float32),
        mesh=mesh,
        compiler_params=cp,
        scratch_types=[
            pltpu.VMEM((nchunks, IDX_W), jnp.int32),
            pltpu.VMEM((nchunks, IDX_W), jnp.int32),
            pltpu.VMEM((bpw, D), jnp.float32),
            pltpu.VMEM((bpw, D), jnp.float32),
            pltpu.VMEM((bpw,), jnp.float32),
            pltpu.VMEM((bpw,), jnp.float32),
            pltpu.VMEM((bpw,), jnp.float32),
            pltpu.VMEM((LANES,), jnp.float32),
            pltpu.SemaphoreType.DMA,
        ],
    )
    def mf_kernel(uid_hbm, iid_hbm, uf_hbm, if_hbm, ub_hbm, ib_hbm, gb_hbm,
                  out_hbm, uid_v, iid_v, urows, irows, ubv, ibv, outv, gbv,
                  sem):
        wid = lax.axis_index("s") * NC + lax.axis_index("c")
        base = wid * bpw

        pltpu.sync_copy(uid_hbm.at[pl.ds(wid * nchunks, nchunks)], uid_v)
        pltpu.sync_copy(iid_hbm.at[pl.ds(wid * nchunks, nchunks)], iid_v)
        pltpu.sync_copy(gb_hbm, gbv)

        copies = []
        for j in range(nchunks):
            rows = pl.ds(j * IDX_W, IDX_W)
            copies.append(pltpu.async_copy(
                uf_hbm.at[uid_v.at[j]], urows.at[rows], sem))
            copies.append(pltpu.async_copy(
                if_hbm.at[iid_v.at[j]], irows.at[rows], sem))
            copies.append(pltpu.async_copy(
                ub_hbm.at[uid_v.at[j]], ubv.at[rows], sem))
            copies.append(pltpu.async_copy(
                ib_hbm.at[iid_v.at[j]], ibv.at[rows], sem))
        for cp in copies:
            cp.wait()

        gb = gbv[...]
        iota16 = lax.iota(jnp.int32, LANES)

        @pl.loop(0, bpw // LANES)
        def _(g):
            sl = pl.ds(g * LANES, LANES)
            row_idx = iota16 + g * LANES
            acc = ubv[sl] + ibv[sl] + gb
            for d in range(D):
                col_idx = jnp.full((LANES,), d, jnp.int32)
                pu = plsc.load_gather(urows, [row_idx, col_idx])
                pv = plsc.load_gather(irows, [row_idx, col_idx])
                acc = acc + pu * pv
            outv[sl] = acc

        pltpu.sync_copy(outv, out_hbm.at[pl.ds(base, bpw)])

    return mf_kernel(uids, iids, user_factors, item_factors, ub_flat,
                     ib_flat, gb_vec)
